# baseline (speedup 1.0000x reference)
import jax
import jax.numpy as jnp
from jax.experimental import pallas as pl
from jax.experimental.pallas import tpu as pltpu


def _copy_kernel(starts_ref, curr_ref, hist_ref, out_ref):
    del starts_ref
    k = pl.program_id(2)

    @pl.when(k == 0)
    def _():
        out_ref[...] = curr_ref[...]

    @pl.when(k != 0)
    def _():
        out_ref[...] = hist_ref[0]


def kernel(feat_curr, feat_history, record_len):
    B, H, C, Hh, W = feat_history.shape
    starts = (jnp.cumsum(record_len) - record_len).astype(jnp.int32)

    CS = C // 2
    grid_spec = pltpu.PrefetchScalarGridSpec(
        num_scalar_prefetch=1,
        grid=(B, 2, H),
        in_specs=[
            pl.BlockSpec(
                (1, CS, Hh, W), lambda bs, c, k, starts: (starts[bs], c, 0, 0)
            ),
            pl.BlockSpec(
                (1, 1, CS, Hh, W),
                lambda bs, c, k, starts: (bs, jnp.maximum(k, 1), c, 0, 0),
            ),
        ],
        out_specs=pl.BlockSpec(
            (1, CS, Hh, W), lambda bs, c, k, starts: (bs * H + k, c, 0, 0)
        ),
    )

    feat_final = pl.pallas_call(
        _copy_kernel,
        grid_spec=grid_spec,
        out_shape=jax.ShapeDtypeStruct((B * H, C, Hh, W), feat_curr.dtype),
    )(starts, feat_curr, feat_history)

    offset_loss = jnp.zeros((1,), dtype=feat_final.dtype)
    return (feat_final, offset_loss)


# speedup vs baseline: 1.2505x; 1.2505x over previous
import jax
import jax.numpy as jnp
from jax.experimental import pallas as pl
from jax.experimental.pallas import tpu as pltpu

_RING = 8


def _copy_body(starts_ref, curr_ref, hist_ref, out_ref, buf, in_sem, out_sem):
    B, H = hist_ref.shape[0], hist_ref.shape[1]
    n = B * H

    def src_at(i):
        bs, k = divmod(i, H)
        if k == 0:
            return curr_ref.at[pl.ds(starts_ref[bs], 1)]
        return hist_ref.at[bs, pl.ds(k, 1)]

    def start_in(i):
        pltpu.make_async_copy(src_at(i), buf.at[pl.ds(i % _RING, 1)], in_sem.at[i]).start()

    def wait_in(i):
        pltpu.make_async_copy(src_at(i), buf.at[pl.ds(i % _RING, 1)], in_sem.at[i]).wait()

    def start_out(i):
        pltpu.make_async_copy(
            buf.at[pl.ds(i % _RING, 1)], out_ref.at[pl.ds(i, 1)], out_sem.at[i]
        ).start()

    def wait_out(i):
        pltpu.make_async_copy(
            buf.at[pl.ds(i % _RING, 1)], out_ref.at[pl.ds(i, 1)], out_sem.at[i]
        ).wait()

    for i in range(_RING):
        start_in(i)
    for i in range(n):
        wait_in(i)
        start_out(i)
        j = i + _RING
        if j < n:
            wait_out(j - _RING)
            start_in(j)
    for i in range(n - _RING, n):
        wait_out(i)


def kernel(feat_curr, feat_history, record_len):
    B, H, C, Hh, W = feat_history.shape
    starts = (jnp.cumsum(record_len) - record_len).astype(jnp.int32)

    feat_final = pl.pallas_call(
        _copy_body,
        in_specs=[
            pl.BlockSpec(memory_space=pltpu.SMEM),
            pl.BlockSpec(memory_space=pltpu.MemorySpace.HBM),
            pl.BlockSpec(memory_space=pltpu.MemorySpace.HBM),
        ],
        out_specs=pl.BlockSpec(memory_space=pltpu.MemorySpace.HBM),
        out_shape=jax.ShapeDtypeStruct((B * H, C, Hh, W), feat_curr.dtype),
        scratch_shapes=[
            pltpu.VMEM((_RING, C, Hh, W), feat_curr.dtype),
            pltpu.SemaphoreType.DMA((B * H,)),
            pltpu.SemaphoreType.DMA((B * H,)),
        ],
    )(starts, feat_curr, feat_history)

    offset_loss = jnp.zeros((1,), dtype=feat_final.dtype)
    return (feat_final, offset_loss)
